# baseline (speedup 1.0000x reference)
import jax
import jax.numpy as jnp
from jax.experimental import pallas as pl
from jax.experimental.pallas import tpu as pltpu

B, N, DIN, DOUT, K = 32, 256, 256, 128, 8
CPK = DOUT // K


def _body(x_ref, adj_ref, w_ref, bias_ref, out_ref, h_ref):
    k = pl.program_id(1)

    @pl.when(k == 0)
    def _compute_h():
        h = jnp.dot(x_ref[0], w_ref[...], preferred_element_type=jnp.float32)
        h_ref[...] = jnp.maximum(h + bias_ref[...], 0.0)

    res = jnp.dot(adj_ref[0, 0], h_ref[...],
                  preferred_element_type=jnp.float32)
    lane_group = jax.lax.broadcasted_iota(jnp.int32, (N, DOUT), 1) // CPK
    out_ref[0] = jnp.where(lane_group == k, res, out_ref[0])


def kernel(node_feats, adj, W, b):
    bias = b.reshape(1, DOUT)
    out = pl.pallas_call(
        _body,
        grid=(B, K),
        in_specs=[
            pl.BlockSpec((1, N, DIN), lambda i, j: (i, 0, 0)),
            pl.BlockSpec((1, 1, N, N), lambda i, j: (i, j, 0, 0)),
            pl.BlockSpec((DIN, DOUT), lambda i, j: (0, 0)),
            pl.BlockSpec((1, DOUT), lambda i, j: (0, 0)),
        ],
        out_specs=pl.BlockSpec((1, N, DOUT), lambda i, j: (i, 0, 0)),
        out_shape=jax.ShapeDtypeStruct((B, N, DOUT), jnp.float32),
        scratch_shapes=[pltpu.VMEM((N, DOUT), jnp.float32)],
        compiler_params=pltpu.CompilerParams(
            dimension_semantics=("arbitrary", "arbitrary"),
        ),
    )(node_feats, adj, W, bias)
    return out


# speedup vs baseline: 3.8945x; 3.8945x over previous
import jax
import jax.numpy as jnp
from jax.experimental import pallas as pl

B, N, DIN, DOUT, K = 32, 256, 256, 128, 8
CPK = DOUT // K


def _body(x_ref, adj_ref, w_ref, bias_ref, out_ref):
    h = jnp.dot(x_ref[0], w_ref[...], preferred_element_type=jnp.float32)
    h = jnp.maximum(h + bias_ref[...], 0.0)
    lane_group = jax.lax.broadcasted_iota(jnp.int32, (N, DOUT), 1) // CPK
    acc = jnp.zeros((N, DOUT), jnp.float32)
    for k in range(K):
        res = jnp.dot(adj_ref[0, k], h, preferred_element_type=jnp.float32)
        acc = acc + jnp.where(lane_group == k, res, 0.0)
    out_ref[0] = acc


def kernel(node_feats, adj, W, b):
    bias = b.reshape(1, DOUT)
    out = pl.pallas_call(
        _body,
        grid=(B,),
        in_specs=[
            pl.BlockSpec((1, N, DIN), lambda i: (i, 0, 0)),
            pl.BlockSpec((1, K, N, N), lambda i: (i, 0, 0, 0)),
            pl.BlockSpec((DIN, DOUT), lambda i: (0, 0)),
            pl.BlockSpec((1, DOUT), lambda i: (0, 0)),
        ],
        out_specs=pl.BlockSpec((1, N, DOUT), lambda i: (i, 0, 0)),
        out_shape=jax.ShapeDtypeStruct((B, N, DOUT), jnp.float32),
    )(node_feats, adj, W, bias)
    return out
